# baseline (speedup 1.0000x reference)
import jax
import jax.numpy as jnp
from jax.experimental import pallas as pl
from jax.experimental.pallas import tpu as pltpu

VOCAB = 7
NAUX = 5
HID = 768


def _body(ids_ref, a0, a1, a2, a3, a4, out_ref):
    G = out_ref.shape[0]
    i = pl.program_id(0)
    ids = jax.lax.bitcast_convert_type(ids_ref[:], jnp.int32)
    ids3 = ids[:, :, None]
    col = jax.lax.broadcasted_iota(jnp.int32, (G, 128, 128), 2)
    acc = (col == ids3).astype(jnp.float32)
    for j, aref in enumerate((a0, a1, a2, a3, a4)):
        acc = jnp.where(col == VOCAB + j, aref[:][:, :, None], acc)
    out_ref[:, :, :128] = acc

    @pl.when(i < 2)
    def _tail():
        out_ref[:, :, 128:] = jnp.zeros((G, 128, HID - 128), jnp.float32)


def kernel(input_ids, aux_features):
    B, T = input_ids.shape
    N = B * T
    NG = N // 128
    ids_f = jax.lax.bitcast_convert_type(
        input_ids.reshape(NG, 128).astype(jnp.int32), jnp.float32
    )
    aux_t = aux_features.reshape(N, NAUX).transpose(1, 0).reshape(NAUX * NG, 128)
    packed = jnp.concatenate([ids_f, aux_t], axis=0)

    G = 32
    S = NG // G
    in_specs = [pl.BlockSpec((G, 128), lambda i: (i, 0))]
    for j in range(NAUX):
        in_specs.append(
            pl.BlockSpec((G, 128), lambda i, j=j: (S * (j + 1) + i, 0))
        )
    out = pl.pallas_call(
        _body,
        grid=(S,),
        in_specs=in_specs,
        out_specs=pl.BlockSpec((G, 128, HID), lambda i: (i, 0, 0)),
        out_shape=jax.ShapeDtypeStruct((NG, 128, HID), jnp.float32),
        compiler_params=pltpu.CompilerParams(
            dimension_semantics=("parallel",),
        ),
    )(*([packed] * (1 + NAUX)))
    return out.reshape(B, T, HID)
